# baseline (speedup 1.0000x reference)
import functools

import jax
import jax.numpy as jnp
import numpy as np
from jax import lax
from jax.experimental import pallas as pl
from jax.experimental.pallas import tpu as pltpu
from jax.experimental.pallas import tpu_sc as plsc

_N = 2048
_B, _C, _H, _W = 16, 256, 64, 64
_P = _H * _W
_M = _B * _P - _N

_NUM_TILES = 32
_SLOTS_PER_TILE = _N // _NUM_TILES
_CHUNK = 128
_CHUNKS_PER_TILE = _SLOTS_PER_TILE * _C // _CHUNK
_INFLIGHT = 8


def _threefry2x32(k1, k2, x0, x1):
    R0, R1 = (13, 15, 26, 6), (17, 29, 16, 24)
    ks0, ks1 = np.uint32(k1), np.uint32(k2)
    ks2 = ks0 ^ ks1 ^ np.uint32(0x1BD11BDA)
    x0 = (x0 + ks0).astype(np.uint32)
    x1 = (x1 + ks1).astype(np.uint32)

    def rounds(x0, x1, rots):
        for r in rots:
            x0 = (x0 + x1).astype(np.uint32)
            x1 = ((x1 << np.uint32(r)) | (x1 >> np.uint32(32 - r))).astype(np.uint32)
            x1 = x0 ^ x1
        return x0, x1

    sched = [(R0, ks1, ks2), (R1, ks2, ks0), (R0, ks0, ks1),
             (R1, ks1, ks2), (R0, ks2, ks0)]
    for i, (rots, ka, kb) in enumerate(sched):
        x0, x1 = rounds(x0, x1, rots)
        x0 = (x0 + ka).astype(np.uint32)
        x1 = (x1 + kb + np.uint32(i + 1)).astype(np.uint32)
    return x0, x1


def _uniform_63488() -> np.ndarray:
    o0, o1 = _threefry2x32(np.uint32(0), np.uint32(42),
                           np.array([0], np.uint32), np.array([7], np.uint32))
    b0, b1 = _threefry2x32(o0[0], o1[0],
                           np.zeros(_M, np.uint32), np.arange(_M, dtype=np.uint32))
    bits = b0 ^ b1
    u = ((bits >> np.uint32(9)) | np.uint32(0x3F800000)).view(np.float32)
    return np.maximum(np.float32(0.0), u - np.float32(1.0))


def _element_indices() -> np.ndarray:
    u = _uniform_63488()
    i_vals = (_N + np.arange(_M)).astype(np.float32)
    idx = np.floor(u * (i_vals + 1.0)).astype(np.int32)
    valid = idx < _N
    step = np.arange(_M, dtype=np.int32)
    last = np.full((_N,), -1, dtype=np.int32)
    np.maximum.at(last, idx[valid], step[valid])
    src_row = np.where(last >= 0, _N + last, np.arange(_N, dtype=np.int32))
    b = src_row // _P
    p = src_row % _P
    c = np.arange(_C, dtype=np.int32)
    elem = (b * (_C * _P) + p)[:, None] + c[None, :] * _P
    return np.ascontiguousarray(elem.reshape(_NUM_TILES, _CHUNK, _CHUNK))


_ELEM_IDX = _element_indices()


def _gather_body(flat_hbm, idx_hbm, out_hbm, idx_v, out_v, sem):
    wid = lax.axis_index("s") * 2 + lax.axis_index("c")
    pltpu.sync_copy(idx_hbm.at[wid], idx_v)

    for j in range(_INFLIGHT):
        pltpu.async_copy(flat_hbm.at[idx_v.at[j]], out_v.at[j], sem)

    @pl.loop(0, _CHUNKS_PER_TILE - _INFLIGHT)
    def _(j):
        pltpu.make_async_copy(flat_hbm.at[idx_v.at[j]], out_v.at[j], sem).wait()
        pltpu.async_copy(
            flat_hbm.at[idx_v.at[j + _INFLIGHT]], out_v.at[j + _INFLIGHT], sem
        )

    for j in range(_CHUNKS_PER_TILE - _INFLIGHT, _CHUNKS_PER_TILE):
        pltpu.make_async_copy(flat_hbm.at[idx_v.at[j]], out_v.at[j], sem).wait()

    pltpu.sync_copy(out_v, out_hbm.at[wid])


@functools.cache
def _build_gather():
    return pl.kernel(
        _gather_body,
        out_type=jax.ShapeDtypeStruct((_NUM_TILES, _CHUNK, _CHUNK), jnp.float32),
        mesh=plsc.VectorSubcoreMesh(core_axis_name="c", subcore_axis_name="s"),
        scratch_types=[
            pltpu.VMEM((_CHUNK, _CHUNK), jnp.int32),
            pltpu.VMEM((_CHUNK, _CHUNK), jnp.float32),
            pltpu.SemaphoreType.DMA,
        ],
    )


def kernel(samples):
    flat = jax.lax.stop_gradient(samples).reshape(-1)
    idx = jnp.asarray(_ELEM_IDX)
    out = _build_gather()(flat, idx)
    return out.reshape(_N, _C)


# speedup vs baseline: 1.0534x; 1.0534x over previous
import functools

import jax
import jax.numpy as jnp
import numpy as np
from jax import lax
from jax.experimental import pallas as pl
from jax.experimental.pallas import tpu as pltpu
from jax.experimental.pallas import tpu_sc as plsc

_N = 2048
_B, _C, _H, _W = 16, 256, 64, 64
_P = _H * _W
_M = _B * _P - _N

_NUM_TILES = 32
_SLOTS_PER_TILE = _N // _NUM_TILES
_CHUNK = 128
_CHUNKS_PER_TILE = _SLOTS_PER_TILE * _C // _CHUNK
_INFLIGHT = 8


def _threefry2x32(k1, k2, x0, x1):
    R0, R1 = (13, 15, 26, 6), (17, 29, 16, 24)
    ks0, ks1 = np.uint32(k1), np.uint32(k2)
    ks2 = ks0 ^ ks1 ^ np.uint32(0x1BD11BDA)
    x0 = (x0 + ks0).astype(np.uint32)
    x1 = (x1 + ks1).astype(np.uint32)

    def rounds(x0, x1, rots):
        for r in rots:
            x0 = (x0 + x1).astype(np.uint32)
            x1 = ((x1 << np.uint32(r)) | (x1 >> np.uint32(32 - r))).astype(np.uint32)
            x1 = x0 ^ x1
        return x0, x1

    sched = [(R0, ks1, ks2), (R1, ks2, ks0), (R0, ks0, ks1),
             (R1, ks1, ks2), (R0, ks2, ks0)]
    for i, (rots, ka, kb) in enumerate(sched):
        x0, x1 = rounds(x0, x1, rots)
        x0 = (x0 + ka).astype(np.uint32)
        x1 = (x1 + kb + np.uint32(i + 1)).astype(np.uint32)
    return x0, x1


def _uniform_63488() -> np.ndarray:
    o0, o1 = _threefry2x32(np.uint32(0), np.uint32(42),
                           np.array([0], np.uint32), np.array([7], np.uint32))
    b0, b1 = _threefry2x32(o0[0], o1[0],
                           np.zeros(_M, np.uint32), np.arange(_M, dtype=np.uint32))
    bits = b0 ^ b1
    u = ((bits >> np.uint32(9)) | np.uint32(0x3F800000)).view(np.float32)
    return np.maximum(np.float32(0.0), u - np.float32(1.0))


def _element_indices() -> np.ndarray:
    u = _uniform_63488()
    i_vals = (_N + np.arange(_M)).astype(np.float32)
    idx = np.floor(u * (i_vals + 1.0)).astype(np.int32)
    valid = idx < _N
    step = np.arange(_M, dtype=np.int32)
    last = np.full((_N,), -1, dtype=np.int32)
    np.maximum.at(last, idx[valid], step[valid])
    src_row = np.where(last >= 0, _N + last, np.arange(_N, dtype=np.int32))
    b = src_row // _P
    p = src_row % _P
    c = np.arange(_C, dtype=np.int32)
    elem = (b * (_C * _P) + p)[:, None] + c[None, :] * _P
    return np.ascontiguousarray(elem.reshape(_NUM_TILES, _CHUNK, _CHUNK))


_ELEM_IDX = _element_indices()


def _gather_body(flat_hbm, idx_hbm, out_hbm, idx_v, out_v, sem):
    wid = lax.axis_index("s") * 2 + lax.axis_index("c")
    pltpu.sync_copy(idx_hbm.at[wid], idx_v)

    @pl.loop(0, _CHUNKS_PER_TILE)
    def _(j):
        pltpu.async_copy(flat_hbm.at[idx_v.at[j]], out_v.at[j], sem)

    @pl.loop(0, _CHUNKS_PER_TILE)
    def _(j):
        pltpu.make_async_copy(flat_hbm.at[idx_v.at[j]], out_v.at[j], sem).wait()

    pltpu.sync_copy(out_v, out_hbm.at[wid])


@functools.cache
def _build_gather():
    return pl.kernel(
        _gather_body,
        out_type=jax.ShapeDtypeStruct((_NUM_TILES, _CHUNK, _CHUNK), jnp.float32),
        mesh=plsc.VectorSubcoreMesh(core_axis_name="c", subcore_axis_name="s"),
        scratch_types=[
            pltpu.VMEM((_CHUNK, _CHUNK), jnp.int32),
            pltpu.VMEM((_CHUNK, _CHUNK), jnp.float32),
            pltpu.SemaphoreType.DMA,
        ],
    )


def kernel(samples):
    flat = jax.lax.stop_gradient(samples).reshape(-1)
    idx = jnp.asarray(_ELEM_IDX)
    out = _build_gather()(flat, idx)
    return out.reshape(_N, _C)


# speedup vs baseline: 9.4890x; 9.0077x over previous
import functools

import jax
import jax.numpy as jnp
import numpy as np
from jax import lax
from jax.experimental import pallas as pl
from jax.experimental.pallas import tpu as pltpu
from jax.experimental.pallas import tpu_sc as plsc

_N = 2048
_B, _C, _H, _W = 16, 256, 64, 64
_P = _H * _W
_M = _B * _P - _N

_NUM_TILES = 32
_ROWS_PER_TILE = _N // _NUM_TILES


def _threefry2x32(k1, k2, x0, x1):
    R0, R1 = (13, 15, 26, 6), (17, 29, 16, 24)
    ks0, ks1 = np.uint32(k1), np.uint32(k2)
    ks2 = ks0 ^ ks1 ^ np.uint32(0x1BD11BDA)
    x0 = (x0 + ks0).astype(np.uint32)
    x1 = (x1 + ks1).astype(np.uint32)

    def rounds(x0, x1, rots):
        for r in rots:
            x0 = (x0 + x1).astype(np.uint32)
            x1 = ((x1 << np.uint32(r)) | (x1 >> np.uint32(32 - r))).astype(np.uint32)
            x1 = x0 ^ x1
        return x0, x1

    sched = [(R0, ks1, ks2), (R1, ks2, ks0), (R0, ks0, ks1),
             (R1, ks1, ks2), (R0, ks2, ks0)]
    for i, (rots, ka, kb) in enumerate(sched):
        x0, x1 = rounds(x0, x1, rots)
        x0 = (x0 + ka).astype(np.uint32)
        x1 = (x1 + kb + np.uint32(i + 1)).astype(np.uint32)
    return x0, x1


def _uniform_63488() -> np.ndarray:
    o0, o1 = _threefry2x32(np.uint32(0), np.uint32(42),
                           np.array([0], np.uint32), np.array([7], np.uint32))
    b0, b1 = _threefry2x32(o0[0], o1[0],
                           np.zeros(_M, np.uint32), np.arange(_M, dtype=np.uint32))
    bits = b0 ^ b1
    u = ((bits >> np.uint32(9)) | np.uint32(0x3F800000)).view(np.float32)
    return np.maximum(np.float32(0.0), u - np.float32(1.0))


def _row_indices() -> np.ndarray:
    u = _uniform_63488()
    i_vals = (_N + np.arange(_M)).astype(np.float32)
    idx = np.floor(u * (i_vals + 1.0)).astype(np.int32)
    valid = idx < _N
    step = np.arange(_M, dtype=np.int32)
    last = np.full((_N,), -1, dtype=np.int32)
    np.maximum.at(last, idx[valid], step[valid])
    src_row = np.where(last >= 0, _N + last, np.arange(_N, dtype=np.int32))
    return np.ascontiguousarray(src_row.reshape(_NUM_TILES, _ROWS_PER_TILE))


_ROW_IDS = _row_indices()


def _gather_body(flat_hbm, idx_hbm, out_hbm, idx_v, out_v, sem):
    wid = lax.axis_index("s") * 2 + lax.axis_index("c")
    pltpu.sync_copy(idx_hbm.at[wid], idx_v)
    pltpu.async_copy(flat_hbm.at[idx_v], out_v, sem).wait()
    pltpu.sync_copy(out_v, out_hbm.at[pl.ds(wid * _ROWS_PER_TILE, _ROWS_PER_TILE)])


@functools.cache
def _build_gather():
    return pl.kernel(
        _gather_body,
        out_type=jax.ShapeDtypeStruct((_N, _C), jnp.float32),
        mesh=plsc.VectorSubcoreMesh(core_axis_name="c", subcore_axis_name="s"),
        scratch_types=[
            pltpu.VMEM((_ROWS_PER_TILE,), jnp.int32),
            pltpu.VMEM((_ROWS_PER_TILE, _C), jnp.float32),
            pltpu.SemaphoreType.DMA,
        ],
        compiler_params=pltpu.CompilerParams(use_tc_tiling_on_sc=True),
    )


def kernel(samples):
    flat = jnp.transpose(jax.lax.stop_gradient(samples), (0, 2, 3, 1))
    flat = flat.reshape(_B * _P, _C)
    return _build_gather()(flat, jnp.asarray(_ROW_IDS))
